# baseline (speedup 1.0000x reference)
import functools

import jax
import jax.numpy as jnp
from jax import lax
from jax.experimental import pallas as pl
from jax.experimental.pallas import tpu as pltpu
from jax.experimental.pallas import tpu_sc as plsc

B, H, W, C = 2, 224, 224, 96
C2 = 2 * C
BC = B * C
HW = H * W

ROW_TILE = 8

NW = 32
PLANES_PER_W = BC // NW
ROWS_PER_CHUNK = 16
CHUNK = ROWS_PER_CHUNK * W
NCHUNKS = H // ROWS_PER_CHUNK
VECS_PER_ROW = W // 16


def _conv_body(xa, xb, xc, w, b, o):
    acc = jnp.broadcast_to(b[0][None, :], (ROW_TILE * W, C2))
    xs = (xa, xb, xc)
    for dy in range(3):
        xr = xs[dy][0]
        for dx in range(3):
            blk = xr[:, dx:dx + W, :].reshape(ROW_TILE * W, C)
            acc = acc + jnp.dot(blk, w[dy, dx],
                                preferred_element_type=jnp.float32)
    o[0] = acc.reshape(ROW_TILE, W, C2)


def _conv_offsets(x, W_conv, b_conv):
    x_pad = jnp.pad(x, ((0, 0), (1, 1), (1, 1), (0, 0)))
    xa = x_pad[:, 0:H, :, :]
    xb = x_pad[:, 1:H + 1, :, :]
    xc = x_pad[:, 2:H + 2, :, :]
    shifted_spec = pl.BlockSpec((1, ROW_TILE, W + 2, C),
                                lambda bi, ti: (bi, ti, 0, 0))
    return pl.pallas_call(
        _conv_body,
        grid=(B, H // ROW_TILE),
        in_specs=[
            shifted_spec, shifted_spec, shifted_spec,
            pl.BlockSpec((3, 3, C, C2), lambda bi, ti: (0, 0, 0, 0)),
            pl.BlockSpec((1, C2), lambda bi, ti: (0, 0)),
        ],
        out_specs=pl.BlockSpec((1, ROW_TILE, W, C2),
                               lambda bi, ti: (bi, ti, 0, 0)),
        out_shape=jax.ShapeDtypeStruct((B, H, W, C2), jnp.float32),
    )(xa, xb, xc, W_conv, b_conv.reshape(1, C2))


@functools.lru_cache(maxsize=1)
def _build_sc_sample():
    mesh = plsc.VectorSubcoreMesh(core_axis_name="c", subcore_axis_name="s")
    return functools.partial(
        pl.kernel,
        mesh=mesh,
        out_type=jax.ShapeDtypeStruct((BC, HW), jnp.float32),
        compiler_params=pltpu.CompilerParams(needs_layout_passes=False),
        scratch_types=[
            pltpu.VMEM((HW,), jnp.float32),
            pltpu.VMEM((CHUNK,), jnp.float32),
            pltpu.VMEM((CHUNK,), jnp.float32),
            pltpu.VMEM((CHUNK,), jnp.float32),
        ],
    )(_sc_sample_body)


def _sc_sample_body(xp_hbm, off0_hbm, off1_hbm, out_hbm, plane_v, o0_v, o1_v, res_v):
    wid = lax.axis_index("s") * 2 + lax.axis_index("c")
    lim = jnp.float32(H - 1)
    xiota = lax.iota(jnp.int32, 16).astype(jnp.float32)

    def plane_body(j, _):
        p = wid * PLANES_PER_W + j
        pltpu.sync_copy(xp_hbm.at[p], plane_v)

        def chunk_body(ci, _):
            base = ci * CHUNK
            pltpu.sync_copy(off0_hbm.at[p, pl.ds(base, CHUNK)], o0_v)
            pltpu.sync_copy(off1_hbm.at[p, pl.ds(base, CHUNK)], o1_v)

            def row_body(r, _):
                y = ci * ROWS_PER_CHUNK + r
                yf = y.astype(jnp.float32)
                for v in range(VECS_PER_ROW):
                    s = r * W + v * 16
                    cy = o0_v[pl.ds(s, 16)] + yf
                    cx = o1_v[pl.ds(s, 16)] + (xiota + jnp.float32(v * 16))
                    cy = jnp.minimum(jnp.maximum(cy, 0.0), lim)
                    cx = jnp.minimum(jnp.maximum(cx, 0.0), lim)
                    iy0 = cy.astype(jnp.int32)
                    ix0 = cx.astype(jnp.int32)
                    fy0 = iy0.astype(jnp.float32)
                    fx0 = ix0.astype(jnp.float32)
                    wy = cy - fy0
                    wx = cx - fx0
                    iy1 = jnp.where(cy > fy0, iy0 + 1, iy0)
                    ix1 = jnp.where(cx > fx0, ix0 + 1, ix0)
                    r0 = iy0 * W
                    r1 = iy1 * W
                    v_lt = plsc.load_gather(plane_v, [r0 + ix0])
                    v_rt = plsc.load_gather(plane_v, [r1 + ix0])
                    v_lb = plsc.load_gather(plane_v, [r0 + ix1])
                    v_rb = plsc.load_gather(plane_v, [r1 + ix1])
                    vt = v_lt + (v_rt - v_lt) * wy
                    vb = v_lb + (v_rb - v_lb) * wy
                    res_v[pl.ds(s, 16)] = vt + (vb - vt) * wx
                return 0

            lax.fori_loop(0, ROWS_PER_CHUNK, row_body, 0)
            pltpu.sync_copy(res_v, out_hbm.at[p, pl.ds(base, CHUNK)])
            return 0

        lax.fori_loop(0, NCHUNKS, chunk_body, 0)
        return 0

    lax.fori_loop(0, PLANES_PER_W, plane_body, 0)


def kernel(x, W_conv, b_conv):
    conv = _conv_offsets(x, W_conv, b_conv)
    offs = conv.transpose(0, 3, 1, 2).reshape(BC, HW, 2)
    off0 = offs[..., 0]
    off1 = offs[..., 1]
    x_bc = x.transpose(0, 3, 1, 2).reshape(BC, HW)
    planes = _build_sc_sample()(x_bc, off0, off1)
    return planes.reshape(B, C, H, W).transpose(0, 2, 3, 1)


# speedup vs baseline: 7.2834x; 7.2834x over previous
import functools

import jax
import jax.numpy as jnp
from jax import lax
from jax.experimental import pallas as pl
from jax.experimental.pallas import tpu as pltpu
from jax.experimental.pallas import tpu_sc as plsc

B, H, W, C = 2, 224, 224, 96
C2 = 2 * C
BC = B * C
HW = H * W

ROW_TILE = 8
NT = H // ROW_TILE
PIX = ROW_TILE * W
HPIX = PIX // 2

NW = 32
PLANES_PER_W = BC // NW
ROWS_PER_CHUNK = 16
CHUNK = ROWS_PER_CHUNK * W
NCHUNKS = H // ROWS_PER_CHUNK
VECS_PER_ROW = W // 16


def _conv_body(x_ref, w_ref, b_ref, o0, o1, xt):
    t = pl.program_id(1)
    r0 = t * ROW_TILE
    acc = jnp.broadcast_to(b_ref[0][None, :], (PIX, C2))
    for dy in range(3):
        rows = x_ref[0, pl.ds(r0 + dy, ROW_TILE), :, :]
        for dx in range(3):
            blk = rows[:, dx:dx + W, :].reshape(PIX, C)
            acc = acc + jnp.dot(blk, w_ref[dy, dx],
                                preferred_element_type=jnp.float32)
    acc2 = acc.reshape(HPIX, 2, C2)
    even = acc2[:, 0, :].T.reshape(C, 2, HPIX)
    odd = acc2[:, 1, :].T.reshape(C, 2, HPIX)
    o0[0, :, :, :] = even
    o1[0, :, :, :] = odd
    xmid = x_ref[0, pl.ds(r0 + 1, ROW_TILE), pl.ds(1, W), :]
    xt[0, :, :] = xmid.reshape(PIX, C).T


def _conv_offsets(x, W_conv, b_conv):
    x_pad = jnp.pad(x, ((0, 0), (1, 1), (1, 1), (0, 0)))
    off_spec = pl.BlockSpec((1, C, 2, HPIX), lambda bi, ti: (bi, 0, 0, ti))
    off_shape = jax.ShapeDtypeStruct((B, C, 2, NT * HPIX), jnp.float32)
    return pl.pallas_call(
        _conv_body,
        grid=(B, NT),
        in_specs=[
            pl.BlockSpec((1, H + 2, W + 2, C), lambda bi, ti: (bi, 0, 0, 0)),
            pl.BlockSpec((3, 3, C, C2), lambda bi, ti: (0, 0, 0, 0)),
            pl.BlockSpec((1, C2), lambda bi, ti: (0, 0)),
        ],
        out_specs=[
            off_spec,
            off_spec,
            pl.BlockSpec((1, C, PIX), lambda bi, ti: (bi, 0, ti)),
        ],
        out_shape=[
            off_shape,
            off_shape,
            jax.ShapeDtypeStruct((B, C, HW), jnp.float32),
        ],
        compiler_params=pltpu.CompilerParams(
            vmem_limit_bytes=100 * 1024 * 1024),
    )(x_pad, W_conv, b_conv.reshape(1, C2))


def _transpose_body(i_ref, o_ref):
    o_ref[0, 0, :, :] = i_ref[0, :, :].T


def _to_nhwc(planes):
    return pl.pallas_call(
        _transpose_body,
        grid=(B, NT),
        in_specs=[pl.BlockSpec((1, C, PIX), lambda bi, ti: (bi, 0, ti))],
        out_specs=pl.BlockSpec((1, 1, PIX, C), lambda bi, ti: (bi, ti, 0, 0)),
        out_shape=jax.ShapeDtypeStruct((B, NT, PIX, C), jnp.float32),
    )(planes)


@functools.lru_cache(maxsize=1)
def _build_sc_sample():
    mesh = plsc.VectorSubcoreMesh(core_axis_name="c", subcore_axis_name="s")
    return functools.partial(
        pl.kernel,
        mesh=mesh,
        out_type=jax.ShapeDtypeStruct((BC, HW), jnp.float32),
        compiler_params=pltpu.CompilerParams(needs_layout_passes=False),
        scratch_types=[
            pltpu.VMEM((HW,), jnp.float32),
            pltpu.VMEM((CHUNK,), jnp.float32),
            pltpu.VMEM((CHUNK,), jnp.float32),
            pltpu.VMEM((CHUNK,), jnp.float32),
        ],
    )(_sc_sample_body)


def _sc_sample_body(xp_hbm, off0_hbm, off1_hbm, out_hbm, plane_v, o0_v, o1_v, res_v):
    wid = lax.axis_index("s") * 2 + lax.axis_index("c")
    lim = jnp.float32(H - 1)
    xiota = lax.iota(jnp.int32, 16).astype(jnp.float32)

    def plane_body(j, _):
        p = wid * PLANES_PER_W + j
        pltpu.sync_copy(xp_hbm.at[p], plane_v)

        def chunk_body(ci, _):
            base = ci * CHUNK
            pltpu.sync_copy(off0_hbm.at[p, pl.ds(base, CHUNK)], o0_v)
            pltpu.sync_copy(off1_hbm.at[p, pl.ds(base, CHUNK)], o1_v)

            def row_body(r, _):
                y = ci * ROWS_PER_CHUNK + r
                yf = y.astype(jnp.float32)
                for v in range(VECS_PER_ROW):
                    s = r * W + v * 16
                    cy = o0_v[pl.ds(s, 16)] + yf
                    cx = o1_v[pl.ds(s, 16)] + (xiota + jnp.float32(v * 16))
                    cy = jnp.minimum(jnp.maximum(cy, 0.0), lim)
                    cx = jnp.minimum(jnp.maximum(cx, 0.0), lim)
                    iy0 = cy.astype(jnp.int32)
                    ix0 = cx.astype(jnp.int32)
                    fy0 = iy0.astype(jnp.float32)
                    fx0 = ix0.astype(jnp.float32)
                    wy = cy - fy0
                    wx = cx - fx0
                    iy1 = jnp.where(cy > fy0, iy0 + 1, iy0)
                    ix1 = jnp.where(cx > fx0, ix0 + 1, ix0)
                    r0 = iy0 * W
                    r1 = iy1 * W
                    v_lt = plsc.load_gather(plane_v, [r0 + ix0])
                    v_rt = plsc.load_gather(plane_v, [r1 + ix0])
                    v_lb = plsc.load_gather(plane_v, [r0 + ix1])
                    v_rb = plsc.load_gather(plane_v, [r1 + ix1])
                    vt = v_lt + (v_rt - v_lt) * wy
                    vb = v_lb + (v_rb - v_lb) * wy
                    res_v[pl.ds(s, 16)] = vt + (vb - vt) * wx
                return 0

            lax.fori_loop(0, ROWS_PER_CHUNK, row_body, 0)
            pltpu.sync_copy(res_v, out_hbm.at[p, pl.ds(base, CHUNK)])
            return 0

        lax.fori_loop(0, NCHUNKS, chunk_body, 0)
        return 0

    lax.fori_loop(0, PLANES_PER_W, plane_body, 0)


def kernel(x, W_conv, b_conv):
    o0, o1, xt = _conv_offsets(x, W_conv, b_conv)
    off0 = o0.reshape(BC, HW)
    off1 = o1.reshape(BC, HW)
    x_bc = xt.reshape(BC, HW)
    planes = _build_sc_sample()(x_bc, off0, off1)
    out = _to_nhwc(planes.reshape(B, C, HW))
    return out.reshape(B, H, W, C)


# speedup vs baseline: 9.5016x; 1.3045x over previous
import functools

import numpy as np

import jax
import jax.numpy as jnp
from jax import lax
from jax.experimental import pallas as pl
from jax.experimental.pallas import tpu as pltpu
from jax.experimental.pallas import tpu_sc as plsc

B, H, W, C = 2, 224, 224, 96
C2 = 2 * C
BC = B * C
HW = H * W

ROW_TILE = 8
NT = H // ROW_TILE
PIX = ROW_TILE * W
HPIX = PIX // 2
HALF = HW // 2

NW = 32
PLANES_PER_W = BC // NW
NCHUNKS = 8
CHUNK = HW // NCHUNKS
ROWS_PER_CHUNK = CHUNK // W
VECS_PER_ROW = W // 16


def _grid_consts():
    l = np.arange(PIX)
    even = (l % 2 == 0)
    g = np.zeros((2, PIX), np.float32)
    for half in range(2):
        g[half] = np.where(even, half * (HALF // W) + (l // 2) // W,
                           (l // 2) % W)
    m = np.broadcast_to(even.astype(np.float32), (2, PIX)).copy()
    return g, m


_G_BASE, _G_TMASK = _grid_consts()


def _conv_body(x_ref, w_ref, b_ref, g_ref, m_ref, co, xt):
    t = pl.program_id(1)
    r0 = t * ROW_TILE
    acc = jnp.broadcast_to(b_ref[0][:, None], (C2, PIX))
    for dy in range(3):
        rows = x_ref[0, pl.ds(r0 + dy, ROW_TILE), :, :]
        for dx in range(3):
            blk = rows[:, dx:dx + W, :].reshape(PIX, C)
            acc = acc + lax.dot_general(
                w_ref[dy, dx], blk, (((0,), (1,)), ((), ())),
                preferred_element_type=jnp.float32)
    g = g_ref[...] + (t * (ROW_TILE // 2)).astype(jnp.float32) * m_ref[...]
    coords = acc.reshape(C, 2, PIX) + g[None, :, :]
    coords = jnp.minimum(jnp.maximum(coords, 0.0), jnp.float32(H - 1))
    co[0, :, :] = coords.reshape(C2, PIX)
    xmid = x_ref[0, pl.ds(r0 + 1, ROW_TILE), pl.ds(1, W), :]
    xt[0, :, :] = xmid.reshape(PIX, C).T


def _conv_coords(x, W_conv, b_conv):
    x_pad = jnp.pad(x, ((0, 0), (1, 1), (1, 1), (0, 0)))
    return pl.pallas_call(
        _conv_body,
        grid=(B, NT),
        in_specs=[
            pl.BlockSpec((1, H + 2, W + 2, C), lambda bi, ti: (bi, 0, 0, 0)),
            pl.BlockSpec((3, 3, C, C2), lambda bi, ti: (0, 0, 0, 0)),
            pl.BlockSpec((1, C2), lambda bi, ti: (0, 0)),
            pl.BlockSpec((2, PIX), lambda bi, ti: (0, 0)),
            pl.BlockSpec((2, PIX), lambda bi, ti: (0, 0)),
        ],
        out_specs=[
            pl.BlockSpec((1, C2, PIX), lambda bi, ti: (bi, 0, ti)),
            pl.BlockSpec((1, C, PIX), lambda bi, ti: (bi, 0, ti)),
        ],
        out_shape=[
            jax.ShapeDtypeStruct((B, C2, HW), jnp.float32),
            jax.ShapeDtypeStruct((B, C, HW), jnp.float32),
        ],
        compiler_params=pltpu.CompilerParams(
            vmem_limit_bytes=100 * 1024 * 1024),
    )(x_pad, W_conv, b_conv.reshape(1, C2), _G_BASE, _G_TMASK)


def _transpose_body(i_ref, o_ref):
    o_ref[0] = i_ref[0].T.reshape(ROW_TILE, W, C)


def _to_nhwc(planes):
    return pl.pallas_call(
        _transpose_body,
        grid=(B, NT),
        in_specs=[pl.BlockSpec((1, C, PIX), lambda bi, ti: (bi, 0, ti))],
        out_specs=pl.BlockSpec((1, ROW_TILE, W, C),
                               lambda bi, ti: (bi, ti, 0, 0)),
        out_shape=jax.ShapeDtypeStruct((B, H, W, C), jnp.float32),
    )(planes)


@functools.lru_cache(maxsize=1)
def _build_sc_sample():
    mesh = plsc.VectorSubcoreMesh(core_axis_name="c", subcore_axis_name="s")
    return functools.partial(
        pl.kernel,
        mesh=mesh,
        out_type=jax.ShapeDtypeStruct((BC, HW), jnp.float32),
        compiler_params=pltpu.CompilerParams(needs_layout_passes=False),
        scratch_types=[
            pltpu.VMEM((HW,), jnp.float32),
            pltpu.VMEM((2 * CHUNK,), jnp.float32),
            pltpu.VMEM((CHUNK,), jnp.float32),
        ],
    )(_sc_sample_body)


def _sc_sample_body(xp_hbm, co_hbm, out_hbm, plane_v, c_v, res_v):
    wid = lax.axis_index("s") * 2 + lax.axis_index("c")
    iota = lax.iota(jnp.int32, 16)
    ev = iota * 2

    def plane_body(j, _):
        p = wid * PLANES_PER_W + j
        b = p // C
        ci = p - b * C
        pltpu.sync_copy(xp_hbm.at[p], plane_v)

        def chunk_body(k, _):
            half = k // (NCHUNKS // 2)
            col0 = (k - half * (NCHUNKS // 2)) * (2 * CHUNK)
            pltpu.sync_copy(
                co_hbm.at[b, 2 * ci + half, pl.ds(col0, 2 * CHUNK)], c_v)

            def row_body(r, _):
                for v in range(VECS_PER_ROW):
                    q = r * W + v * 16
                    cidx = ev + (2 * q)
                    cy = plsc.load_gather(c_v, [cidx])
                    cx = plsc.load_gather(c_v, [cidx + 1])
                    iy0 = cy.astype(jnp.int32)
                    ix0 = cx.astype(jnp.int32)
                    wy = cy - iy0.astype(jnp.float32)
                    wx = cx - ix0.astype(jnp.float32)
                    ady = jnp.where(iy0 < H - 1, W, 0)
                    adx = jnp.where(ix0 < W - 1, 1, 0)
                    i_lt = iy0 * W + ix0
                    i_rt = i_lt + ady
                    v_lt = plsc.load_gather(plane_v, [i_lt])
                    v_rt = plsc.load_gather(plane_v, [i_rt])
                    v_lb = plsc.load_gather(plane_v, [i_lt + adx])
                    v_rb = plsc.load_gather(plane_v, [i_rt + adx])
                    vt = v_lt + (v_rt - v_lt) * wy
                    vb = v_lb + (v_rb - v_lb) * wy
                    res_v[pl.ds(q, 16)] = vt + (vb - vt) * wx
                return 0

            lax.fori_loop(0, ROWS_PER_CHUNK, row_body, 0, unroll=2)
            pltpu.sync_copy(res_v, out_hbm.at[p, pl.ds(k * CHUNK, CHUNK)])
            return 0

        lax.fori_loop(0, NCHUNKS, chunk_body, 0)
        return 0

    lax.fori_loop(0, PLANES_PER_W, plane_body, 0)


def kernel(x, W_conv, b_conv):
    co, xt = _conv_coords(x, W_conv, b_conv)
    x_bc = xt.reshape(BC, HW)
    planes = _build_sc_sample()(x_bc, co)
    return _to_nhwc(planes.reshape(B, C, HW))


# speedup vs baseline: 17.2451x; 1.8150x over previous
import functools

import numpy as np

import jax
import jax.numpy as jnp
from jax import lax
from jax.experimental import pallas as pl
from jax.experimental.pallas import tpu as pltpu
from jax.experimental.pallas import tpu_sc as plsc

B, H, W, C = 2, 224, 224, 96
C2 = 2 * C
BC = B * C
HW = H * W

ROW_TILE = 8
NT = H // ROW_TILE
PIX = ROW_TILE * W
HPIX = PIX // 2
HALF = HW // 2

NW = 32
PLANES_PER_W = BC // NW
NCHUNKS = 8
CHUNK = HW // NCHUNKS
ROWS_PER_CHUNK = CHUNK // W
VECS_PER_ROW = W // 16


def _grid_consts():
    l = np.arange(PIX)
    even = (l % 2 == 0)
    g = np.zeros((2, PIX), np.float32)
    for half in range(2):
        g[half] = np.where(even, half * (HALF // W) + (l // 2) // W,
                           (l // 2) % W)
    m = np.broadcast_to(even.astype(np.float32), (2, PIX)).copy()
    return g, m


_G_BASE, _G_TMASK = _grid_consts()


def _conv_body(x_ref, w_ref, b_ref, g_ref, m_ref, co, xt):
    t = pl.program_id(1)
    r0 = t * ROW_TILE
    acc = jnp.broadcast_to(b_ref[0][:, None], (C2, PIX))
    for dy in range(3):
        rows = x_ref[0, pl.ds(r0 + dy, ROW_TILE), :, :]
        for dx in range(3):
            blk = rows[:, dx:dx + W, :].reshape(PIX, C)
            acc = acc + lax.dot_general(
                w_ref[dy, dx], blk, (((0,), (1,)), ((), ())),
                preferred_element_type=jnp.float32)
    g = g_ref[...] + (t * (ROW_TILE // 2)).astype(jnp.float32) * m_ref[...]
    coords = acc.reshape(C, 2, PIX) + g[None, :, :]
    coords = jnp.minimum(jnp.maximum(coords, 0.0), jnp.float32(H - 1))
    co[0, :, :] = coords.reshape(C2, PIX)
    xmid = x_ref[0, pl.ds(r0 + 1, ROW_TILE), pl.ds(1, W), :]
    xt[0, :, :] = xmid.reshape(PIX, C).T


def _conv_coords(x, W_conv, b_conv):
    x_pad = jnp.pad(x, ((0, 0), (1, 1), (1, 1), (0, 0)))
    return pl.pallas_call(
        _conv_body,
        grid=(B, NT),
        in_specs=[
            pl.BlockSpec((1, H + 2, W + 2, C), lambda bi, ti: (bi, 0, 0, 0)),
            pl.BlockSpec((3, 3, C, C2), lambda bi, ti: (0, 0, 0, 0)),
            pl.BlockSpec((1, C2), lambda bi, ti: (0, 0)),
            pl.BlockSpec((2, PIX), lambda bi, ti: (0, 0)),
            pl.BlockSpec((2, PIX), lambda bi, ti: (0, 0)),
        ],
        out_specs=[
            pl.BlockSpec((1, C2, PIX), lambda bi, ti: (bi, 0, ti)),
            pl.BlockSpec((1, C, PIX), lambda bi, ti: (bi, 0, ti)),
        ],
        out_shape=[
            jax.ShapeDtypeStruct((B, C2, HW), jnp.float32),
            jax.ShapeDtypeStruct((B, C, HW), jnp.float32),
        ],
        compiler_params=pltpu.CompilerParams(
            vmem_limit_bytes=100 * 1024 * 1024),
    )(x_pad, W_conv, b_conv.reshape(1, C2), _G_BASE, _G_TMASK)


def _transpose_body(i_ref, o_ref):
    o_ref[0] = i_ref[0].T.reshape(ROW_TILE, W, C)


def _to_nhwc(planes):
    return pl.pallas_call(
        _transpose_body,
        grid=(B, NT),
        in_specs=[pl.BlockSpec((1, C, PIX), lambda bi, ti: (bi, 0, ti))],
        out_specs=pl.BlockSpec((1, ROW_TILE, W, C),
                               lambda bi, ti: (bi, ti, 0, 0)),
        out_shape=jax.ShapeDtypeStruct((B, H, W, C), jnp.float32),
    )(planes)


@functools.lru_cache(maxsize=1)
def _build_sc_sample():
    mesh = plsc.VectorSubcoreMesh(core_axis_name="c", subcore_axis_name="s")
    return functools.partial(
        pl.kernel,
        mesh=mesh,
        out_type=jax.ShapeDtypeStruct((BC, HW), jnp.float32),
        compiler_params=pltpu.CompilerParams(needs_layout_passes=False),
        scratch_types=[
            pltpu.VMEM((HW,), jnp.float32),
            pltpu.VMEM((2 * CHUNK,), jnp.float32),
            pltpu.VMEM((CHUNK,), jnp.float32),
        ],
    )(_sc_sample_body)


def _sc_sample_body(xp_hbm, co_hbm, out_hbm, plane_v, c_v, res_v):
    wid = lax.axis_index("s") * 2 + lax.axis_index("c")
    iota = lax.iota(jnp.int32, 16)
    ev = iota * 2

    def plane_body(j, _):
        p = wid * PLANES_PER_W + j
        b = p // C
        ci = p - b * C
        pltpu.sync_copy(xp_hbm.at[p], plane_v)

        def chunk_body(k, _):
            half = k // (NCHUNKS // 2)
            col0 = (k - half * (NCHUNKS // 2)) * (2 * CHUNK)
            pltpu.sync_copy(
                co_hbm.at[b, 2 * ci + half, pl.ds(col0, 2 * CHUNK)], c_v)

            @functools.partial(
                plsc.parallel_loop, 0, ROWS_PER_CHUNK, unroll=2)
            def row_body(r):
                for v in range(VECS_PER_ROW):
                    q = r * W + v * 16
                    cidx = ev + (2 * q)
                    cy = plsc.load_gather(c_v, [cidx])
                    cx = plsc.load_gather(c_v, [cidx + 1])
                    iy0 = cy.astype(jnp.int32)
                    ix0 = cx.astype(jnp.int32)
                    wy = cy - iy0.astype(jnp.float32)
                    wx = cx - ix0.astype(jnp.float32)
                    ady = jnp.where(iy0 < H - 1, W, 0)
                    adx = jnp.where(ix0 < W - 1, 1, 0)
                    i_lt = iy0 * W + ix0
                    i_rt = i_lt + ady
                    v_lt = plsc.load_gather(plane_v, [i_lt])
                    v_rt = plsc.load_gather(plane_v, [i_rt])
                    v_lb = plsc.load_gather(plane_v, [i_lt + adx])
                    v_rb = plsc.load_gather(plane_v, [i_rt + adx])
                    vt = v_lt + (v_rt - v_lt) * wy
                    vb = v_lb + (v_rb - v_lb) * wy
                    res_v[pl.ds(q, 16)] = vt + (vb - vt) * wx

            pltpu.sync_copy(res_v, out_hbm.at[p, pl.ds(k * CHUNK, CHUNK)])
            return 0

        lax.fori_loop(0, NCHUNKS, chunk_body, 0)
        return 0

    lax.fori_loop(0, PLANES_PER_W, plane_body, 0)


def kernel(x, W_conv, b_conv):
    co, xt = _conv_coords(x, W_conv, b_conv)
    x_bc = xt.reshape(BC, HW)
    planes = _build_sc_sample()(x_bc, co)
    return _to_nhwc(planes.reshape(B, C, HW))
